# baseline (speedup 1.0000x reference)
import functools

import jax
import jax.numpy as jnp
from jax.experimental import pallas as pl
from jax.experimental.pallas import tpu as pltpu

_N_SAE = 8
_D_DATA = 768
_D_DICT = 1536
_TOKENS = 2048
_BT = 1024


def _fused_body(x_ref, gt_ref, we_ref, be_ref, wd_ref, bd_ref, out_ref):
    s = pl.program_id(1)
    x = x_ref[...]
    bd = bd_ref[0, 0, :]
    g = gt_ref[0, 0, :]
    xc = x - bd[None, :]
    m = jnp.dot(xc, we_ref[0], preferred_element_type=jnp.float32)
    a = jax.nn.relu(m + be_ref[0, 0, :][None, :])
    ga = g[:, None] * a
    d = jnp.dot(ga, wd_ref[0], preferred_element_type=jnp.float32)
    contrib = d + (g != 0.0).astype(jnp.float32)[:, None] * bd[None, :]

    @pl.when(s == 0)
    def _():
        out_ref[...] = contrib

    @pl.when(s > 0)
    def _():
        out_ref[...] = out_ref[...] + contrib


@jax.jit
def kernel(x, gate, W_enc, b_enc, W_dec, b_dec):
    gate_t = gate.T.reshape(_N_SAE, 1, _TOKENS)
    b_enc3 = b_enc.reshape(_N_SAE, 1, _D_DICT)
    b_dec3 = b_dec.reshape(_N_SAE, 1, _D_DATA)
    grid = (_TOKENS // _BT, _N_SAE)
    out = pl.pallas_call(
        _fused_body,
        grid=grid,
        in_specs=[
            pl.BlockSpec((_BT, _D_DATA), lambda i, s: (i, 0)),
            pl.BlockSpec((1, 1, _BT), lambda i, s: (s, 0, i)),
            pl.BlockSpec((1, _D_DATA, _D_DICT), lambda i, s: (s, 0, 0)),
            pl.BlockSpec((1, 1, _D_DICT), lambda i, s: (s, 0, 0)),
            pl.BlockSpec((1, _D_DICT, _D_DATA), lambda i, s: (s, 0, 0)),
            pl.BlockSpec((1, 1, _D_DATA), lambda i, s: (s, 0, 0)),
        ],
        out_specs=pl.BlockSpec((_BT, _D_DATA), lambda i, s: (i, 0)),
        out_shape=jax.ShapeDtypeStruct((_TOKENS, _D_DATA), jnp.float32),
        compiler_params=pltpu.CompilerParams(
            dimension_semantics=("parallel", "arbitrary"),
        ),
    )(x, gate_t, W_enc, b_enc3, W_dec, b_dec3)
    return out
